# baseline (speedup 1.0000x reference)
import functools

import jax
import jax.numpy as jnp
from jax import lax
from jax.experimental import pallas as pl
from jax.experimental.pallas import tpu as pltpu
from jax.experimental.pallas import tpu_sc as plsc

_N = 10000
_E = 320000
_H = 4
_DI = 128
_DP = 32
_LR = 0.2
_PC = 64
_VW = 80
_NT = 16
_SB = 80
_EPT = _E // _NT
_NB = _EPT // _SB
_RPT = _N // _NT
_FC = 125
_TB = 1000


def _tc_body(x_ref, w_ref, a1_ref, a2_ref, h_ref, s1_ref):
    c = pl.program_id(0)
    h = jnp.dot(x_ref[...], w_ref[...], preferred_element_type=jnp.float32)
    s1 = jnp.dot(h, a1_ref[...], preferred_element_type=jnp.float32)
    s2 = jnp.dot(h, a2_ref[...], preferred_element_type=jnp.float32)
    hh = jnp.where(c == 0, h[:, :_PC], h[:, _PC:])
    s2s = jnp.where(c == 0, s2[:, :2], s2[:, 2:])
    h_ref[...] = jnp.concatenate(
        [hh, s2s, jnp.zeros((_TB, _VW - _PC - 2), jnp.float32)], axis=1)
    s1_ref[...] = jnp.where(c == 0, s1[:, :2], s1[:, 2:])


_tc_call = pl.pallas_call(
    _tc_body,
    grid=(2, _N // _TB),
    in_specs=[
        pl.BlockSpec((_TB, _DI), lambda c, j: (j, 0)),
        pl.BlockSpec((_DI, _DI), lambda c, j: (0, 0)),
        pl.BlockSpec((_DI, _H), lambda c, j: (0, 0)),
        pl.BlockSpec((_DI, _H), lambda c, j: (0, 0)),
    ],
    out_specs=[
        pl.BlockSpec((_TB, _VW), lambda c, j: (c * (_N // _TB) + j, 0)),
        pl.BlockSpec((_TB, 2), lambda c, j: (c * (_N // _TB) + j, 0)),
    ],
    out_shape=[
        jax.ShapeDtypeStruct((2 * _N, _VW), jnp.float32),
        jax.ShapeDtypeStruct((2 * _N, 2), jnp.float32),
    ],
)


@functools.partial(
    pl.kernel,
    out_type=jax.ShapeDtypeStruct((_N, _DI), jnp.float32),
    mesh=plsc.VectorSubcoreMesh(core_axis_name="c", subcore_axis_name="s",
                                num_cores=2, num_subcores=_NT),
    compiler_params=pltpu.CompilerParams(use_tc_tiling_on_sc=False,
                                         needs_layout_passes=False),
    scratch_types=[
        pltpu.VMEM((2 * _N,), jnp.float32),
        pltpu.VMEM((_SB,), jnp.int32),
        pltpu.VMEM((_SB,), jnp.int32),
        pltpu.VMEM((_SB,), jnp.int32),
        pltpu.VMEM((_SB,), jnp.int32),
        pltpu.VMEM((1, _SB), jnp.int32),
        pltpu.VMEM((1, _SB), jnp.int32),
        pltpu.VMEM((1, _SB), jnp.int32),
        pltpu.VMEM((1, _SB), jnp.int32),
        pltpu.VMEM((1, _SB), jnp.int32),
        pltpu.VMEM((1, _SB), jnp.int32),
        pltpu.VMEM((1, _SB), jnp.int32),
        pltpu.VMEM((1, _SB), jnp.int32),
        pltpu.VMEM((_SB, _VW), jnp.float32),
        pltpu.VMEM((_SB, _VW), jnp.float32),
        pltpu.VMEM((_SB, _VW), jnp.float32),
        pltpu.VMEM((_SB, _VW), jnp.float32),
        pltpu.VMEM((_SB, _VW), jnp.float32),
        pltpu.VMEM((_SB, _VW), jnp.float32),
        pltpu.VMEM((_FC, _VW), jnp.float32),
        pltpu.VMEM((_FC, _PC), jnp.float32),
        pltpu.VMEM_SHARED((_N, _VW), jnp.float32),
        pltpu.SemaphoreType.DMA,
        pltpu.SemaphoreType.DMA,
        pltpu.SemaphoreType.DMA,
        pltpu.SemaphoreType.DMA,
        pltpu.SemaphoreType.DMA,
        pltpu.SemaphoreType.DMA,
        pltpu.SemaphoreType.DMA,
        pltpu.SemaphoreType.DMA,
    ],
)
def _sc_call(hcat, s1flat, src_hbm, dst_hbm, out_hbm,
             s1_vm, pfs0, pfs1, pfd0, pfd1, src0, src1,
             srcs0, srcs1, srcs2, srcs3, dsti0, dsti1, hg0, hg1,
             val0, val1, val2, val3, accf, outf, acc_sh,
             sg0, sg1, ss0, ss1, ss2, ss3, si0, si1):
    c = lax.axis_index("c")
    s = lax.axis_index("s")
    cn = c * _N
    iota = lax.iota(jnp.int32, 16)
    zf = jnp.zeros((16,), jnp.float32)
    c64 = jnp.full((16,), _PC, jnp.int32)
    c65 = jnp.full((16,), _PC + 1, jnp.int32)

    pfss = (pfs0, pfs1)
    pfds = (pfd0, pfd1)
    srcs = (src0, src1)
    srcss = (srcs0, srcs1, srcs2, srcs3)
    dstis = (dsti0, dsti1)
    hgs = (hg0, hg1)
    vals = (val0, val1, val2, val3)
    sgs = (sg0, sg1)
    sss = (ss0, ss1, ss2, ss3)
    sis = (si0, si1)

    def _zrow(r, carry):
        for q in range(_VW // 16):
            val0[r, pl.ds(q * 16, 16)] = zf
            val1[r, pl.ds(q * 16, 16)] = zf
            val2[r, pl.ds(q * 16, 16)] = zf
            val3[r, pl.ds(q * 16, 16)] = zf
        return carry
    lax.fori_loop(0, _SB, _zrow, 0)
    row0 = s * _RPT
    for k in range(_RPT // _SB):
        pltpu.sync_copy(val0, acc_sh.at[pl.ds(row0 + k * _SB, _SB)])
    pltpu.sync_copy(val0.at[pl.ds(0, _RPT % _SB)],
                    acc_sh.at[pl.ds(row0 + (_RPT // _SB) * _SB, _RPT % _SB)])

    pltpu.sync_copy(s1flat.at[pl.ds(cn * 2, 2 * _N)], s1_vm)

    plsc.subcore_barrier()

    rb0 = (s * _EPT) // _SB

    def _fire_idx(par, row):
        off = jnp.minimum(row, _E // _SB - 1) * _SB
        pltpu.async_copy(src_hbm.at[pl.ds(off, _SB)], pfss[par], sis[par])
        pltpu.async_copy(dst_hbm.at[pl.ds(off, _SB)], pfds[par], sis[par])

    def _wait_idx(par):
        pltpu.make_async_copy(src_hbm.at[pl.ds(0, _SB)], pfss[par],
                              sis[par]).wait()
        pltpu.make_async_copy(dst_hbm.at[pl.ds(0, _SB)], pfds[par],
                              sis[par]).wait()

    def _promote_idx(par):
        for q in range(_SB // 16):
            srcs[par][0, pl.ds(q * 16, 16)] = pfss[par][pl.ds(q * 16, 16)]
            dstis[par][0, pl.ds(q * 16, 16)] = (
                pfds[par][pl.ds(q * 16, 16)] + cn)

    def _fire_gather(par):
        pltpu.async_copy(hcat.at[dstis[par].at[0]], hgs[par], sgs[par])

    def _wait_gather(par):
        pltpu.make_async_copy(hcat.at[pl.ds(0, _SB)], hgs[par],
                              sgs[par]).wait()

    def _fire_scatter(par, r):
        for q in range(_SB // 16):
            srcss[r][0, pl.ds(q * 16, 16)] = srcs[par][0, pl.ds(q * 16, 16)]
        pltpu.async_copy(vals[r], acc_sh.at[srcss[r].at[0]], sss[r],
                         add=True)

    def _wait_scatter(r):
        pltpu.make_async_copy(hcat.at[pl.ds(0, _SB)], vals[r],
                              sss[r]).wait()

    def _compute(par, r):
        hgp, valp, srcp = hgs[par], vals[r], srcs[par]

        def _grp(q):
            srcv = srcp[0, pl.ds(q * 16, 16)]
            rows = q * 16 + iota
            s2a = plsc.load_gather(hgp, [rows, c64])
            s2b = plsc.load_gather(hgp, [rows, c65])
            src2 = srcv * 2
            s1a = plsc.load_gather(s1_vm, [src2])
            s1b = plsc.load_gather(s1_vm, [src2 + 1])
            a0 = s1a + s2a
            a1 = s1b + s2b
            p0 = jnp.exp(jnp.maximum(a0, a0 * _LR))
            p1 = jnp.exp(jnp.maximum(a1, a1 * _LR))
            plsc.store_scatter(valp, [rows, c64], p0)
            plsc.store_scatter(valp, [rows, c65], p1)
        plsc.parallel_loop(0, _SB // 16)(_grp)

        def _scale(e):
            pv = valp[e, pl.ds(_PC, 16)]
            pe0 = pv[0]
            pe1 = pv[1]
            for jj in range(_DP // 16):
                valp[e, pl.ds(jj * 16, 16)] = (
                    hgp[e, pl.ds(jj * 16, 16)] * pe0)
            for jj in range(_DP // 16):
                valp[e, pl.ds(_DP + jj * 16, 16)] = (
                    hgp[e, pl.ds(_DP + jj * 16, 16)] * pe1)
        plsc.parallel_loop(0, _SB)(_scale)

    def _step(par, r, pf_row, wait_sc):
        if wait_sc:
            _wait_scatter(r)
        _wait_gather(par)
        _compute(par, r)
        _fire_scatter(par, r)
        _wait_idx(par)
        _promote_idx(par)
        _fire_gather(par)
        _fire_idx(par, pf_row)

    for par in range(2):
        _fire_idx(par, rb0 + par)
    for par in range(2):
        _wait_idx(par)
        _promote_idx(par)
        _fire_gather(par)
        _fire_idx(par, rb0 + par + 2)
    for b in range(4):
        _step(b % 2, b, rb0 + b + 4, False)

    def _body(t, carry):
        b = 4 * t + 4
        for i in range(4):
            _step(i % 2, i, rb0 + b + i + 4, True)
        return carry
    lax.fori_loop(0, (_NB - 4 - 2) // 4, _body, 0)

    for b in range(_NB - 2, _NB):
        _step(b % 2, b % 4, rb0 + b + 4, True)

    for r in range(4):
        _wait_scatter(r)
    for par in range(2):
        _wait_gather(par)
        _wait_idx(par)

    plsc.subcore_barrier()

    def _chunk(k, carry):
        r0 = row0 + k * _FC
        pltpu.sync_copy(acc_sh.at[pl.ds(r0, _FC)], accf)

        def _fin(r, carry2):
            rv = r + jnp.zeros((16,), jnp.int32)
            d0 = plsc.load_gather(accf, [rv, c64])
            d1 = plsc.load_gather(accf, [rv, c65])
            i0 = 1.0 / (d0 + 1e-16)
            i1 = 1.0 / (d1 + 1e-16)
            for jj in range(_PC // 16):
                seg = accf[r, pl.ds(jj * 16, 16)]
                outf[r, pl.ds(jj * 16, 16)] = seg * (i0 if jj < 2 else i1)
            return carry2
        lax.fori_loop(0, _FC, _fin, 0)
        pltpu.sync_copy(outf,
                        out_hbm.at[pl.ds(r0, _FC), pl.ds(c * _PC, _PC)])
        return carry
    lax.fori_loop(0, _RPT // _FC, _chunk, 0)


@jax.jit
def kernel(input, adj, W, a):
    src = adj[0]
    dst = adj[1]
    w_cat = jnp.transpose(W, (1, 0, 2)).reshape(_DI, _H * _DP)
    a1 = jnp.zeros((_H * _DP, _H), jnp.float32)
    a2 = jnp.zeros((_H * _DP, _H), jnp.float32)
    for hd in range(_H):
        a1 = a1.at[hd * _DP:(hd + 1) * _DP, hd].set(a[hd, 0, :_DP])
        a2 = a2.at[hd * _DP:(hd + 1) * _DP, hd].set(a[hd, 0, _DP:])
    hcat, s1cat = _tc_call(input, w_cat, a1, a2)
    s1flat = s1cat.reshape(4 * _N)
    return _sc_call(hcat, s1flat, src, dst)


# speedup vs baseline: 1.0175x; 1.0175x over previous
import functools

import jax
import jax.numpy as jnp
from jax import lax
from jax.experimental import pallas as pl
from jax.experimental.pallas import tpu as pltpu
from jax.experimental.pallas import tpu_sc as plsc

_N = 10000
_E = 320000
_H = 4
_DI = 128
_DP = 32
_LR = 0.2
_PC = 64
_VW = 80
_NT = 16
_SB = 80
_EPT = _E // _NT
_NB = _EPT // _SB
_RPT = _N // _NT
_FC = 125
_TB = 1000


def _tc_body(x_ref, w_ref, a_ref, h_ref, s1_ref):
    x = x_ref[...]
    h0 = jnp.dot(x, w_ref[0], preferred_element_type=jnp.float32)
    h1 = jnp.dot(x, w_ref[1], preferred_element_type=jnp.float32)
    hh = jnp.concatenate([h0, h1], axis=1)
    z32 = jnp.zeros((_DP,), jnp.float32)
    a1p = jnp.stack(
        [jnp.concatenate([a_ref[0, 0, :_DP], z32]),
         jnp.concatenate([z32, a_ref[1, 0, :_DP]])], axis=1)
    a2p = jnp.stack(
        [jnp.concatenate([a_ref[0, 0, _DP:], z32]),
         jnp.concatenate([z32, a_ref[1, 0, _DP:]])], axis=1)
    s1p = jnp.dot(hh, a1p, preferred_element_type=jnp.float32)
    s2p = jnp.dot(hh, a2p, preferred_element_type=jnp.float32)
    h_ref[...] = jnp.concatenate(
        [hh, s2p, jnp.zeros((_TB, _VW - _PC - 2), jnp.float32)], axis=1)
    s1_ref[...] = s1p


_tc_call = pl.pallas_call(
    _tc_body,
    grid=(2, _N // _TB),
    in_specs=[
        pl.BlockSpec((_TB, _DI), lambda c, j: (j, 0)),
        pl.BlockSpec((2, _DI, _DP), lambda c, j: (c, 0, 0)),
        pl.BlockSpec((2, 1, 2 * _DP), lambda c, j: (c, 0, 0)),
    ],
    out_specs=[
        pl.BlockSpec((_TB, _VW), lambda c, j: (c * (_N // _TB) + j, 0)),
        pl.BlockSpec((_TB, 2), lambda c, j: (c * (_N // _TB) + j, 0)),
    ],
    out_shape=[
        jax.ShapeDtypeStruct((2 * _N, _VW), jnp.float32),
        jax.ShapeDtypeStruct((2 * _N, 2), jnp.float32),
    ],
)


@functools.partial(
    pl.kernel,
    out_type=jax.ShapeDtypeStruct((_N, _DI), jnp.float32),
    mesh=plsc.VectorSubcoreMesh(core_axis_name="c", subcore_axis_name="s",
                                num_cores=2, num_subcores=_NT),
    compiler_params=pltpu.CompilerParams(use_tc_tiling_on_sc=False,
                                         needs_layout_passes=False),
    scratch_types=[
        pltpu.VMEM((2 * _N,), jnp.float32),
        pltpu.VMEM((_SB,), jnp.int32),
        pltpu.VMEM((_SB,), jnp.int32),
        pltpu.VMEM((_SB,), jnp.int32),
        pltpu.VMEM((_SB,), jnp.int32),
        pltpu.VMEM((1, _SB), jnp.int32),
        pltpu.VMEM((1, _SB), jnp.int32),
        pltpu.VMEM((1, _SB), jnp.int32),
        pltpu.VMEM((1, _SB), jnp.int32),
        pltpu.VMEM((1, _SB), jnp.int32),
        pltpu.VMEM((1, _SB), jnp.int32),
        pltpu.VMEM((1, _SB), jnp.int32),
        pltpu.VMEM((1, _SB), jnp.int32),
        pltpu.VMEM((_SB, _VW), jnp.float32),
        pltpu.VMEM((_SB, _VW), jnp.float32),
        pltpu.VMEM((_SB, _VW), jnp.float32),
        pltpu.VMEM((_SB, _VW), jnp.float32),
        pltpu.VMEM((_SB, _VW), jnp.float32),
        pltpu.VMEM((_SB, _VW), jnp.float32),
        pltpu.VMEM((_FC, _VW), jnp.float32),
        pltpu.VMEM((_FC, _PC), jnp.float32),
        pltpu.VMEM_SHARED((_N, _VW), jnp.float32),
        pltpu.SemaphoreType.DMA,
        pltpu.SemaphoreType.DMA,
        pltpu.SemaphoreType.DMA,
        pltpu.SemaphoreType.DMA,
        pltpu.SemaphoreType.DMA,
        pltpu.SemaphoreType.DMA,
        pltpu.SemaphoreType.DMA,
        pltpu.SemaphoreType.DMA,
    ],
)
def _sc_call(hcat, s1flat, src_hbm, dst_hbm, out_hbm,
             s1_vm, pfs0, pfs1, pfd0, pfd1, src0, src1,
             srcs0, srcs1, srcs2, srcs3, dsti0, dsti1, hg0, hg1,
             val0, val1, val2, val3, accf, outf, acc_sh,
             sg0, sg1, ss0, ss1, ss2, ss3, si0, si1):
    c = lax.axis_index("c")
    s = lax.axis_index("s")
    cn = c * _N
    iota = lax.iota(jnp.int32, 16)
    zf = jnp.zeros((16,), jnp.float32)
    c64 = jnp.full((16,), _PC, jnp.int32)
    c65 = jnp.full((16,), _PC + 1, jnp.int32)

    pfss = (pfs0, pfs1)
    pfds = (pfd0, pfd1)
    srcs = (src0, src1)
    srcss = (srcs0, srcs1, srcs2, srcs3)
    dstis = (dsti0, dsti1)
    hgs = (hg0, hg1)
    vals = (val0, val1, val2, val3)
    sgs = (sg0, sg1)
    sss = (ss0, ss1, ss2, ss3)
    sis = (si0, si1)

    def _zrow(r, carry):
        for q in range(_VW // 16):
            val0[r, pl.ds(q * 16, 16)] = zf
            val1[r, pl.ds(q * 16, 16)] = zf
            val2[r, pl.ds(q * 16, 16)] = zf
            val3[r, pl.ds(q * 16, 16)] = zf
        return carry
    lax.fori_loop(0, _SB, _zrow, 0)
    row0 = s * _RPT
    for k in range(_RPT // _SB):
        pltpu.sync_copy(val0, acc_sh.at[pl.ds(row0 + k * _SB, _SB)])
    pltpu.sync_copy(val0.at[pl.ds(0, _RPT % _SB)],
                    acc_sh.at[pl.ds(row0 + (_RPT // _SB) * _SB, _RPT % _SB)])

    pltpu.sync_copy(s1flat.at[pl.ds(cn * 2, 2 * _N)], s1_vm)

    plsc.subcore_barrier()

    rb0 = (s * _EPT) // _SB

    def _fire_idx(par, row):
        off = jnp.minimum(row, _E // _SB - 1) * _SB
        pltpu.async_copy(src_hbm.at[pl.ds(off, _SB)], pfss[par], sis[par])
        pltpu.async_copy(dst_hbm.at[pl.ds(off, _SB)], pfds[par], sis[par])

    def _wait_idx(par):
        pltpu.make_async_copy(src_hbm.at[pl.ds(0, _SB)], pfss[par],
                              sis[par]).wait()
        pltpu.make_async_copy(dst_hbm.at[pl.ds(0, _SB)], pfds[par],
                              sis[par]).wait()

    def _promote_idx(par):
        for q in range(_SB // 16):
            srcs[par][0, pl.ds(q * 16, 16)] = pfss[par][pl.ds(q * 16, 16)]
            dstis[par][0, pl.ds(q * 16, 16)] = (
                pfds[par][pl.ds(q * 16, 16)] + cn)

    def _fire_gather(par):
        pltpu.async_copy(hcat.at[dstis[par].at[0]], hgs[par], sgs[par])

    def _wait_gather(par):
        pltpu.make_async_copy(hcat.at[pl.ds(0, _SB)], hgs[par],
                              sgs[par]).wait()

    def _fire_scatter(par, r):
        for q in range(_SB // 16):
            srcss[r][0, pl.ds(q * 16, 16)] = srcs[par][0, pl.ds(q * 16, 16)]
        pltpu.async_copy(vals[r], acc_sh.at[srcss[r].at[0]], sss[r],
                         add=True)

    def _wait_scatter(r):
        pltpu.make_async_copy(hcat.at[pl.ds(0, _SB)], vals[r],
                              sss[r]).wait()

    def _compute(par, r):
        hgp, valp, srcp = hgs[par], vals[r], srcs[par]

        def _grp(q):
            srcv = srcp[0, pl.ds(q * 16, 16)]
            rows = q * 16 + iota
            s2a = plsc.load_gather(hgp, [rows, c64])
            s2b = plsc.load_gather(hgp, [rows, c65])
            src2 = srcv * 2
            s1a = plsc.load_gather(s1_vm, [src2])
            s1b = plsc.load_gather(s1_vm, [src2 + 1])
            a0 = s1a + s2a
            a1 = s1b + s2b
            p0 = jnp.exp(jnp.maximum(a0, a0 * _LR))
            p1 = jnp.exp(jnp.maximum(a1, a1 * _LR))
            plsc.store_scatter(valp, [rows, c64], p0)
            plsc.store_scatter(valp, [rows, c65], p1)
        plsc.parallel_loop(0, _SB // 16)(_grp)

        def _scale(e):
            pv = valp[e, pl.ds(_PC, 16)]
            pe0 = pv[0]
            pe1 = pv[1]
            for jj in range(_DP // 16):
                valp[e, pl.ds(jj * 16, 16)] = (
                    hgp[e, pl.ds(jj * 16, 16)] * pe0)
            for jj in range(_DP // 16):
                valp[e, pl.ds(_DP + jj * 16, 16)] = (
                    hgp[e, pl.ds(_DP + jj * 16, 16)] * pe1)
        plsc.parallel_loop(0, _SB)(_scale)

    def _step(par, r, pf_row, wait_sc):
        if wait_sc:
            _wait_scatter(r)
        _wait_gather(par)
        _compute(par, r)
        _fire_scatter(par, r)
        _wait_idx(par)
        _promote_idx(par)
        _fire_gather(par)
        _fire_idx(par, pf_row)

    for par in range(2):
        _fire_idx(par, rb0 + par)
    for par in range(2):
        _wait_idx(par)
        _promote_idx(par)
        _fire_gather(par)
        _fire_idx(par, rb0 + par + 2)
    for b in range(4):
        _step(b % 2, b, rb0 + b + 4, False)

    def _body(t, carry):
        b = 4 * t + 4
        for i in range(4):
            _step(i % 2, i, rb0 + b + i + 4, True)
        return carry
    lax.fori_loop(0, (_NB - 4 - 2) // 4, _body, 0)

    for b in range(_NB - 2, _NB):
        _step(b % 2, b % 4, rb0 + b + 4, True)

    for r in range(4):
        _wait_scatter(r)
    for par in range(2):
        _wait_gather(par)
        _wait_idx(par)

    plsc.subcore_barrier()

    def _chunk(k, carry):
        r0 = row0 + k * _FC
        pltpu.sync_copy(acc_sh.at[pl.ds(r0, _FC)], accf)

        def _fin(r, carry2):
            rv = r + jnp.zeros((16,), jnp.int32)
            d0 = plsc.load_gather(accf, [rv, c64])
            d1 = plsc.load_gather(accf, [rv, c65])
            i0 = 1.0 / (d0 + 1e-16)
            i1 = 1.0 / (d1 + 1e-16)
            for jj in range(_PC // 16):
                seg = accf[r, pl.ds(jj * 16, 16)]
                outf[r, pl.ds(jj * 16, 16)] = seg * (i0 if jj < 2 else i1)
            return carry2
        lax.fori_loop(0, _FC, _fin, 0)
        pltpu.sync_copy(outf,
                        out_hbm.at[pl.ds(r0, _FC), pl.ds(c * _PC, _PC)])
        return carry
    lax.fori_loop(0, _RPT // _FC, _chunk, 0)


@jax.jit
def kernel(input, adj, W, a):
    src = adj[0]
    dst = adj[1]
    hcat, s1cat = _tc_call(input, W, a)
    s1flat = s1cat.reshape(4 * _N)
    return _sc_call(hcat, s1flat, src, dst)
